# baseline (speedup 1.0000x reference)
import functools

import jax
import jax.numpy as jnp
from jax import lax
from jax.experimental import pallas as pl
from jax.experimental.pallas import tpu as pltpu
from jax.experimental.pallas import tpu_sc as plsc

_VOCAB = 1000000
_EMBED = 64
_BATCH = 16384
_C = 5

_NC = 2
_NS = 16
_NW = _NC * _NS
_BPW = _BATCH // _NW
_CB = 256
_NCHUNK = _BPW // _CB
_IW = 128


def _dots_kernel(tt_hbm, ct_hbm, tgt_hbm, ctx_hbm, out_hbm,
                 idx_t, idx_c, rows_t, rows_c, out_v, sem):
    wid = lax.axis_index("s") * _NC + lax.axis_index("c")
    base = wid * _BPW

    for ch in range(_NCHUNK):
        b0 = base + ch * _CB
        pltpu.sync_copy(tgt_hbm.at[pl.ds(b0, _CB)], idx_t)
        pltpu.sync_copy(ctx_hbm.at[pl.ds(b0 * _C, _CB * _C)], idx_c)

        copies = []
        for j in range(_CB // _IW):
            copies.append(pltpu.async_copy(
                tt_hbm.at[idx_t.at[pl.ds(j * _IW, _IW)]],
                rows_t.at[pl.ds(j * _IW, _IW)], sem))
        for j in range(_CB * _C // _IW):
            copies.append(pltpu.async_copy(
                ct_hbm.at[idx_c.at[pl.ds(j * _IW, _IW)]],
                rows_c.at[pl.ds(j * _IW, _IW)], sem))
        for cp in copies:
            cp.wait()

        def bg_body(bg, _):
            bvec = lax.iota(jnp.int32, 16) + bg * 16
            crow = [bvec * _C + c for c in range(_C)]
            acc = [jnp.zeros((16,), jnp.float32) for _ in range(_C)]
            for e in range(_EMBED):
                ecol = jnp.full((16,), e, jnp.int32)
                wv = plsc.load_gather(rows_t, [bvec, ecol])
                for c in range(_C):
                    cv = plsc.load_gather(rows_c, [crow[c], ecol])
                    acc[c] = acc[c] + wv * cv
            for c in range(_C):
                plsc.store_scatter(out_v, [crow[c]], acc[c])
            return _

        lax.fori_loop(0, _CB // 16, bg_body, None)

        pltpu.sync_copy(out_v, out_hbm.at[pl.ds(b0 * _C, _CB * _C)])


@jax.jit
def _run(target, context, target_table, context_table):
    mesh = plsc.VectorSubcoreMesh(core_axis_name="c", subcore_axis_name="s",
                                  num_cores=_NC, num_subcores=_NS)
    k = functools.partial(
        pl.kernel,
        out_type=jax.ShapeDtypeStruct((_BATCH * _C,), jnp.float32),
        mesh=mesh,
        compiler_params=pltpu.CompilerParams(
            needs_layout_passes=False, use_tc_tiling_on_sc=False),
        scratch_types=[
            pltpu.VMEM((_CB,), jnp.int32),
            pltpu.VMEM((_CB * _C,), jnp.int32),
            pltpu.VMEM((_CB, _EMBED), jnp.float32),
            pltpu.VMEM((_CB * _C, _EMBED), jnp.float32),
            pltpu.VMEM((_CB * _C,), jnp.float32),
            pltpu.SemaphoreType.DMA,
        ],
    )(_dots_kernel)
    flat = k(target_table, context_table, target, context.reshape(-1))
    return flat.reshape(_BATCH, _C)


def kernel(target, context, target_table, context_table):
    if target.ndim == 2:
        target = jnp.squeeze(target, axis=1)
    return _run(target.astype(jnp.int32), context.astype(jnp.int32),
                target_table, context_table)
